# baseline (speedup 1.0000x reference)
import math

import jax
import jax.numpy as jnp
from jax.experimental import pallas as pl

_H = 512
_W = 512
_MAX_SEGMENTS = 50
_MIN_PIXELS = 20
_SHADOW_OFFSET = 7.0
_TRANSLATION_STRENGTH = 20.0
_CC_ITERS = 64
_BIG = _H * _W


def _cc_kernel_body(mask_ref, out_ref):
    mask = mask_ref[:]
    row = jax.lax.broadcasted_iota(jnp.int32, (2, _H, _W), 1)
    col = jax.lax.broadcasted_iota(jnp.int32, (2, _H, _W), 2)
    idx = row * _W + col
    lab = jnp.where(mask > 0, idx, _BIG)

    def body(_, lab):
        p = jnp.pad(lab, ((0, 0), (1, 1), (1, 1)), constant_values=_BIG)
        nb = jnp.minimum(
            jnp.minimum(p[:, :-2, 1:-1], p[:, 2:, 1:-1]),
            jnp.minimum(p[:, 1:-1, :-2], p[:, 1:-1, 2:]))
        return jnp.where(mask > 0, jnp.minimum(lab, nb), _BIG)

    lab = jax.lax.fori_loop(0, _CC_ITERS, body, lab)
    out_ref[:] = lab


def _cc_pallas(masks, interpret=False):
    return pl.pallas_call(
        _cc_kernel_body,
        out_shape=jax.ShapeDtypeStruct((2, _H, _W), jnp.int32),
        interpret=interpret,
    )(masks)


def _seg_stats(lab, fiedler, m):
    num = _BIG + 1
    flat = lab.reshape(-1)
    ys, xs = jnp.meshgrid(jnp.arange(_H, dtype=jnp.float32),
                          jnp.arange(_W, dtype=jnp.float32), indexing='ij')
    yf = ys.reshape(-1)
    xf = xs.reshape(-1)
    f = fiedler.reshape(-1)
    mf_ = m.reshape(-1)
    cnt = jax.ops.segment_sum(mf_, flat, num)
    sy = jax.ops.segment_sum(yf * mf_, flat, num)
    sx = jax.ops.segment_sum(xf * mf_, flat, num)
    sf = jax.ops.segment_sum(f * mf_, flat, num)
    sf2 = jax.ops.segment_sum(f * f * mf_, flat, num)
    sr2 = jax.ops.segment_sum((yf * yf + xf * xf) * mf_, flat, num)
    c = jnp.maximum(cnt, 1.0)
    cy = sy / c
    cx = sx / c
    mf = sf / c
    vf = jnp.maximum(sf2 / c - mf * mf, 0.0)
    g2 = jnp.maximum(sr2 / c - (cy * cy + cx * cx), 0.0)
    sig = jnp.stack([mf, jnp.sqrt(vf), jnp.sqrt(cnt) / _H, jnp.sqrt(g2) / _H], axis=-1)
    cnt = cnt.at[num - 1].set(0.0)
    return cnt, cy, cx, mf, sig


def _pipeline(img_A, img_B, fiedler_A, fiedler_B, gate_A, contours_A, interpret=False):
    h, w = _H, _W
    mask_A = jnp.logical_and(contours_A, gate_A < 0.5)
    mask_B = fiedler_B > jnp.mean(fiedler_B)
    masks = jnp.stack([mask_A, mask_B]).astype(jnp.int32)
    labs = _cc_pallas(masks, interpret=interpret)
    lab_A, lab_B = labs[0], labs[1]
    mA = mask_A.astype(jnp.float32)
    mB = mask_B.astype(jnp.float32)
    cntA, cyA, cxA, mfA, sigA = _seg_stats(lab_A, fiedler_A, mA)
    cntB, cyB, cxB, mfB, sigB = _seg_stats(lab_B, fiedler_B, mB)
    selA = jnp.where(cntA >= _MIN_PIXELS, cntA, 0.0)
    selB = jnp.where(cntB >= _MIN_PIXELS, cntB, 0.0)
    valsA, idsA = jax.lax.top_k(selA, _MAX_SEGMENTS)
    valsB, idsB = jax.lax.top_k(selB, _MAX_SEGMENTS)
    validA = valsA >= float(_MIN_PIXELS)
    validB = valsB >= float(_MIN_PIXELS)
    sA = sigA[idsA]
    sB = sigB[idsB]
    dist = jnp.sum((sA[:, None, :] - sB[None, :, :]) ** 2, axis=-1)
    dist = dist + 1e9 * (1.0 - validB.astype(jnp.float32))[None, :]
    match = jnp.argmin(dist, axis=1)
    ys, xs = jnp.meshgrid(jnp.arange(h, dtype=jnp.float32),
                          jnp.arange(w, dtype=jnp.float32), indexing='ij')
    imgBf = img_B.reshape(-1, 3)
    front_num = jnp.zeros((h * w, 3), dtype=jnp.float32)
    front_den = jnp.zeros((h * w,), dtype=jnp.float32)
    shadow_den = jnp.zeros((h * w,), dtype=jnp.float32)
    for i in range(_MAX_SEGMENTS):
        labA_i = idsA[i]
        j = match[i]
        labB_j = idsB[j]
        ok = jnp.logical_and(validA[i], validB[j]).astype(jnp.float32)
        cay = cyA[labA_i]
        cax = cxA[labA_i]
        cby = cyB[labB_j]
        cbx = cxB[labB_j]
        m = (lab_B == labB_j).astype(jnp.float32) * ok
        dy = ys - cby
        dx = xs - cbx
        th = mfA[labA_i] * math.pi
        ty = jnp.round(cay + dx + _TRANSLATION_STRENGTH * jnp.sin(th))
        tx = jnp.round(cax - dy + _TRANSLATION_STRENGTH * jnp.cos(th))
        inb = ((ty >= 0) & (ty <= h - 1) & (tx >= 0) & (tx <= w - 1)).astype(jnp.float32)
        wv = (m * inb).reshape(-1)
        ti = (jnp.clip(ty, 0, h - 1).astype(jnp.int32) * w
              + jnp.clip(tx, 0, w - 1).astype(jnp.int32)).reshape(-1)
        front_num = front_num.at[ti].add(imgBf * wv[:, None])
        front_den = front_den.at[ti].add(wv)
        sy = ty + _SHADOW_OFFSET
        sx2 = tx + _SHADOW_OFFSET
        sin_b = ((sy >= 0) & (sy <= h - 1) & (sx2 >= 0) & (sx2 <= w - 1)).astype(jnp.float32)
        si = (jnp.clip(sy, 0, h - 1).astype(jnp.int32) * w
              + jnp.clip(sx2, 0, w - 1).astype(jnp.int32)).reshape(-1)
        shadow_den = shadow_den.at[si].add((m * sin_b).reshape(-1))
    fm = jnp.clip(front_den, 0.0, 1.0).reshape(h, w, 1)
    fc = (front_num / jnp.maximum(front_den[:, None], 1e-6)).reshape(h, w, 3)
    sm = jnp.clip(shadow_den, 0.0, 1.0).reshape(h, w, 1)
    out = img_A * (1.0 - 0.5 * sm)
    out = out * (1.0 - fm) + fm * fc
    return out


def kernel(img_A, img_B, fiedler_A, fiedler_B, gate_A, contours_A):
    return _pipeline(img_A, img_B, fiedler_A, fiedler_B, gate_A, contours_A)


# speedup vs baseline: 24.4608x; 24.4608x over previous
import math

import jax
import jax.numpy as jnp
from jax.experimental import pallas as pl
from jax.experimental.pallas import tpu as pltpu

_H = 512
_W = 512
_MAX_SEGMENTS = 50
_MIN_PIXELS = 20
_SHADOW_OFFSET = 7.0
_TRANSLATION_STRENGTH = 20.0
_CC_ITERS = 64
_BIG = _H * _W


def _cc_kernel_body(mask_ref, out_ref):
    mask = mask_ref[:]
    row = jax.lax.broadcasted_iota(jnp.int32, (2, _H, _W), 1)
    col = jax.lax.broadcasted_iota(jnp.int32, (2, _H, _W), 2)
    idx = row * _W + col
    lab = jnp.where(mask > 0, idx, _BIG)

    def body(_, lab):
        p = jnp.pad(lab, ((0, 0), (1, 1), (1, 1)), constant_values=_BIG)
        nb = jnp.minimum(
            jnp.minimum(p[:, :-2, 1:-1], p[:, 2:, 1:-1]),
            jnp.minimum(p[:, 1:-1, :-2], p[:, 1:-1, 2:]))
        return jnp.where(mask > 0, jnp.minimum(lab, nb), _BIG)

    lab = jax.lax.fori_loop(0, _CC_ITERS, body, lab)
    out_ref[:] = lab


def _cc_pallas(masks, interpret=False):
    return pl.pallas_call(
        _cc_kernel_body,
        out_shape=jax.ShapeDtypeStruct((2, _H, _W), jnp.int32),
        interpret=interpret,
    )(masks)


def _raster_kernel_body(base4_ref, imgA_ref, labB_ref, scf_ref, sci_ref,
                        out_ref, acc_ref, sh_ref):
    acc_ref[...] = jnp.zeros((4, _H, _W), jnp.float32)
    sh_ref[...] = jnp.zeros((_H, _W), jnp.float32)
    base4 = base4_ref[:]
    labB = labB_ref[:]
    row_iota = jax.lax.broadcasted_iota(jnp.int32, (_H, _W), 0)
    col_iota = jax.lax.broadcasted_iota(jnp.int32, (_H, _W), 1)
    xsf = jax.lax.broadcasted_iota(jnp.int32, (1, _W), 1).astype(jnp.float32)
    ysf = jax.lax.broadcasted_iota(jnp.int32, (_H, 1), 0).astype(jnp.float32)

    def body(i, _):
        labj = sci_ref[0, i]
        cay = scf_ref[0, i]
        cax = scf_ref[1, i]
        cby = scf_ref[2, i]
        cbx = scf_ref[3, i]
        tsin = scf_ref[4, i]
        tcos = scf_ref[5, i]
        okv = scf_ref[6, i]

        dx = xsf - cbx
        ty = jnp.round(cay + dx + tsin)
        dy = ysf - cby
        tx = jnp.round(cax - dy + tcos)
        iy = (ty >= 0) & (ty <= _H - 1)
        ix = (tx >= 0) & (tx <= _W - 1)
        tyi = jnp.clip(ty, 0, _H - 1).astype(jnp.int32)
        txi = jnp.clip(tx, 0, _W - 1).astype(jnp.int32)
        PT = ((row_iota == tyi) & iy).astype(jnp.float32)
        Q = ((col_iota == txi) & ix).astype(jnp.float32)
        sy = ty + _SHADOW_OFFSET
        sx = tx + _SHADOW_OFFSET
        isy = (sy >= 0) & (sy <= _H - 1)
        isx = (sx >= 0) & (sx <= _W - 1)
        syi = jnp.clip(sy, 0, _H - 1).astype(jnp.int32)
        sxi = jnp.clip(sx, 0, _W - 1).astype(jnp.int32)
        PTs = ((row_iota == syi) & isy).astype(jnp.float32)
        Qs = ((col_iota == sxi) & isx).astype(jnp.float32)

        m = (labB == labj).astype(jnp.float32) * okv
        mst = (base4 * m[None, :, :]).reshape(4 * _H, _W)
        big1 = jax.lax.dot_general(PT, mst, (((1,), (1,)), ((), ())),
                                   preferred_element_type=jnp.float32)
        for ch in range(4):
            tmp = big1[:, ch * _H:(ch + 1) * _H]
            acc_ref[ch] += jax.lax.dot_general(
                tmp, Q, (((1,), (0,)), ((), ())),
                preferred_element_type=jnp.float32)
        tmps = jax.lax.dot_general(PTs, m, (((1,), (1,)), ((), ())),
                                   preferred_element_type=jnp.float32)
        sh_ref[...] += jax.lax.dot_general(tmps, Qs, (((1,), (0,)), ((), ())),
                                           preferred_element_type=jnp.float32)
        return 0

    jax.lax.fori_loop(0, _MAX_SEGMENTS, body, 0)

    den = acc_ref[3]
    fm = jnp.clip(den, 0.0, 1.0)
    sm = jnp.clip(sh_ref[...], 0.0, 1.0)
    dsafe = jnp.maximum(den, 1e-6)
    for ch in range(3):
        fc = acc_ref[ch] / dsafe
        o = imgA_ref[ch] * (1.0 - 0.5 * sm)
        out_ref[ch] = o * (1.0 - fm) + fm * fc


def _raster_pallas(base4, imgA3, labB, scf, sci, interpret=False):
    return pl.pallas_call(
        _raster_kernel_body,
        out_shape=jax.ShapeDtypeStruct((3, _H, _W), jnp.float32),
        in_specs=[
            pl.BlockSpec(memory_space=pltpu.VMEM),
            pl.BlockSpec(memory_space=pltpu.VMEM),
            pl.BlockSpec(memory_space=pltpu.VMEM),
            pl.BlockSpec(memory_space=pltpu.SMEM),
            pl.BlockSpec(memory_space=pltpu.SMEM),
        ],
        out_specs=pl.BlockSpec(memory_space=pltpu.VMEM),
        scratch_shapes=[
            pltpu.VMEM((4, _H, _W), jnp.float32),
            pltpu.VMEM((_H, _W), jnp.float32),
        ],
        interpret=interpret,
    )(base4, imgA3, labB, scf, sci)


def _seg_stats(lab, fiedler, m):
    num = _BIG + 1
    flat = lab.reshape(-1)
    ys, xs = jnp.meshgrid(jnp.arange(_H, dtype=jnp.float32),
                          jnp.arange(_W, dtype=jnp.float32), indexing='ij')
    yf = ys.reshape(-1)
    xf = xs.reshape(-1)
    f = fiedler.reshape(-1)
    mf_ = m.reshape(-1)
    cnt = jax.ops.segment_sum(mf_, flat, num)
    sy = jax.ops.segment_sum(yf * mf_, flat, num)
    sx = jax.ops.segment_sum(xf * mf_, flat, num)
    sf = jax.ops.segment_sum(f * mf_, flat, num)
    sf2 = jax.ops.segment_sum(f * f * mf_, flat, num)
    sr2 = jax.ops.segment_sum((yf * yf + xf * xf) * mf_, flat, num)
    c = jnp.maximum(cnt, 1.0)
    cy = sy / c
    cx = sx / c
    mf = sf / c
    vf = jnp.maximum(sf2 / c - mf * mf, 0.0)
    g2 = jnp.maximum(sr2 / c - (cy * cy + cx * cx), 0.0)
    sig = jnp.stack([mf, jnp.sqrt(vf), jnp.sqrt(cnt) / _H, jnp.sqrt(g2) / _H], axis=-1)
    cnt = cnt.at[num - 1].set(0.0)
    return cnt, cy, cx, mf, sig


def _pipeline(img_A, img_B, fiedler_A, fiedler_B, gate_A, contours_A, interpret=False):
    h, w = _H, _W
    mask_A = jnp.logical_and(contours_A, gate_A < 0.5)
    mask_B = fiedler_B > jnp.mean(fiedler_B)
    masks = jnp.stack([mask_A, mask_B]).astype(jnp.int32)
    labs = _cc_pallas(masks, interpret=interpret)
    lab_A, lab_B = labs[0], labs[1]
    mA = mask_A.astype(jnp.float32)
    mB = mask_B.astype(jnp.float32)
    cntA, cyA, cxA, mfA, sigA = _seg_stats(lab_A, fiedler_A, mA)
    cntB, cyB, cxB, mfB, sigB = _seg_stats(lab_B, fiedler_B, mB)
    selA = jnp.where(cntA >= _MIN_PIXELS, cntA, 0.0)
    selB = jnp.where(cntB >= _MIN_PIXELS, cntB, 0.0)
    valsA, idsA = jax.lax.top_k(selA, _MAX_SEGMENTS)
    valsB, idsB = jax.lax.top_k(selB, _MAX_SEGMENTS)
    validA = valsA >= float(_MIN_PIXELS)
    validB = valsB >= float(_MIN_PIXELS)
    sA = sigA[idsA]
    sB = sigB[idsB]
    dist = jnp.sum((sA[:, None, :] - sB[None, :, :]) ** 2, axis=-1)
    dist = dist + 1e9 * (1.0 - validB.astype(jnp.float32))[None, :]
    match = jnp.argmin(dist, axis=1)
    labB_sel = idsB[match]
    okv = (validA & validB[match]).astype(jnp.float32)
    cay = cyA[idsA]
    cax = cxA[idsA]
    cby = cyB[labB_sel]
    cbx = cxB[labB_sel]
    th = mfA[idsA] * math.pi
    tsin = _TRANSLATION_STRENGTH * jnp.sin(th)
    tcos = _TRANSLATION_STRENGTH * jnp.cos(th)
    scf = jnp.stack([cay, cax, cby, cbx, tsin, tcos, okv])
    sci = labB_sel.reshape(1, _MAX_SEGMENTS)
    base4 = jnp.concatenate(
        [jnp.moveaxis(img_B, -1, 0), jnp.ones((1, h, w), jnp.float32)])
    imgA3 = jnp.moveaxis(img_A, -1, 0)
    out3 = _raster_pallas(base4, imgA3, lab_B, scf, sci, interpret=interpret)
    return jnp.moveaxis(out3, 0, -1)


def kernel(img_A, img_B, fiedler_A, fiedler_B, gate_A, contours_A):
    return _pipeline(img_A, img_B, fiedler_A, fiedler_B, gate_A, contours_A)


# speedup vs baseline: 30.2719x; 1.2376x over previous
import math

import jax
import jax.numpy as jnp
from jax.experimental import pallas as pl
from jax.experimental.pallas import tpu as pltpu

_H = 512
_W = 512
_MAX_SEGMENTS = 50
_MIN_PIXELS = 20
_SHADOW_OFFSET = 7.0
_TRANSLATION_STRENGTH = 20.0
_CC_ITERS = 64
_BIG = _H * _W


def _cc_kernel_body(mask_ref, out_ref):
    mask = mask_ref[:]
    row = jax.lax.broadcasted_iota(jnp.int32, (2, _H, _W), 1)
    col = jax.lax.broadcasted_iota(jnp.int32, (2, _H, _W), 2)
    idx = row * _W + col
    lab = jnp.where(mask > 0, idx, _BIG)

    def body(_, lab):
        p = jnp.pad(lab, ((0, 0), (1, 1), (1, 1)), constant_values=_BIG)
        nb = jnp.minimum(
            jnp.minimum(p[:, :-2, 1:-1], p[:, 2:, 1:-1]),
            jnp.minimum(p[:, 1:-1, :-2], p[:, 1:-1, 2:]))
        return jnp.where(mask > 0, jnp.minimum(lab, nb), _BIG)

    lab = jax.lax.fori_loop(0, _CC_ITERS, body, lab)
    out_ref[:] = lab


def _cc_pallas(masks, interpret=False):
    return pl.pallas_call(
        _cc_kernel_body,
        out_shape=jax.ShapeDtypeStruct((2, _H, _W), jnp.int32),
        interpret=interpret,
    )(masks)


def _raster_kernel_body(base4_ref, imgA_ref, labB_ref, scf_ref, sci_ref,
                        out_ref, acc_ref, sh_ref):
    acc_ref[...] = jnp.zeros((4, _H, _W), jnp.float32)
    sh_ref[...] = jnp.zeros((_H, _W), jnp.float32)
    base4 = base4_ref[:].astype(jnp.bfloat16)
    labB = labB_ref[:]
    row_iota = jax.lax.broadcasted_iota(jnp.int32, (_H, _W), 0)
    col_iota = jax.lax.broadcasted_iota(jnp.int32, (_H, _W), 1)
    xsf = jax.lax.broadcasted_iota(jnp.int32, (1, _W), 1).astype(jnp.float32)
    ysf = jax.lax.broadcasted_iota(jnp.int32, (_H, 1), 0).astype(jnp.float32)

    def body(i, _):
        labj = sci_ref[0, i]
        cay = scf_ref[0, i]
        cax = scf_ref[1, i]
        cby = scf_ref[2, i]
        cbx = scf_ref[3, i]
        tsin = scf_ref[4, i]
        tcos = scf_ref[5, i]
        okv = scf_ref[6, i]

        dx = xsf - cbx
        ty = jnp.round(cay + dx + tsin)
        dy = ysf - cby
        tx = jnp.round(cax - dy + tcos)
        iy = (ty >= 0) & (ty <= _H - 1)
        ix = (tx >= 0) & (tx <= _W - 1)
        tyi = jnp.clip(ty, 0, _H - 1).astype(jnp.int32)
        txi = jnp.clip(tx, 0, _W - 1).astype(jnp.int32)
        PT = ((row_iota == tyi) & iy).astype(jnp.bfloat16)
        Q = ((col_iota == txi) & ix).astype(jnp.bfloat16)
        sy = ty + _SHADOW_OFFSET
        sx = tx + _SHADOW_OFFSET
        isy = (sy >= 0) & (sy <= _H - 1)
        isx = (sx >= 0) & (sx <= _W - 1)
        syi = jnp.clip(sy, 0, _H - 1).astype(jnp.int32)
        sxi = jnp.clip(sx, 0, _W - 1).astype(jnp.int32)
        PTs = ((row_iota == syi) & isy).astype(jnp.bfloat16)
        Qs = ((col_iota == sxi) & isx).astype(jnp.bfloat16)

        m = (labB == labj).astype(jnp.bfloat16) * okv.astype(jnp.bfloat16)
        mst = (base4 * m[None, :, :]).reshape(4 * _H, _W)
        big1 = jax.lax.dot_general(PT, mst, (((1,), (1,)), ((), ())),
                                   preferred_element_type=jnp.float32)
        big1 = big1.astype(jnp.bfloat16)
        for ch in range(4):
            tmp = big1[:, ch * _H:(ch + 1) * _H]
            acc_ref[ch] += jax.lax.dot_general(
                tmp, Q, (((1,), (0,)), ((), ())),
                preferred_element_type=jnp.float32)
        tmps = jax.lax.dot_general(PTs, m, (((1,), (1,)), ((), ())),
                                   preferred_element_type=jnp.float32)
        sh_ref[...] += jax.lax.dot_general(
            tmps.astype(jnp.bfloat16), Qs, (((1,), (0,)), ((), ())),
            preferred_element_type=jnp.float32)
        return 0

    jax.lax.fori_loop(0, _MAX_SEGMENTS, body, 0)

    den = acc_ref[3]
    fm = jnp.clip(den, 0.0, 1.0)
    sm = jnp.clip(sh_ref[...], 0.0, 1.0)
    dsafe = jnp.maximum(den, 1e-6)
    for ch in range(3):
        fc = acc_ref[ch] / dsafe
        o = imgA_ref[ch] * (1.0 - 0.5 * sm)
        out_ref[ch] = o * (1.0 - fm) + fm * fc


def _raster_pallas(base4, imgA3, labB, scf, sci, interpret=False):
    return pl.pallas_call(
        _raster_kernel_body,
        out_shape=jax.ShapeDtypeStruct((3, _H, _W), jnp.float32),
        in_specs=[
            pl.BlockSpec(memory_space=pltpu.VMEM),
            pl.BlockSpec(memory_space=pltpu.VMEM),
            pl.BlockSpec(memory_space=pltpu.VMEM),
            pl.BlockSpec(memory_space=pltpu.SMEM),
            pl.BlockSpec(memory_space=pltpu.SMEM),
        ],
        out_specs=pl.BlockSpec(memory_space=pltpu.VMEM),
        scratch_shapes=[
            pltpu.VMEM((4, _H, _W), jnp.float32),
            pltpu.VMEM((_H, _W), jnp.float32),
        ],
        interpret=interpret,
    )(base4, imgA3, labB, scf, sci)


def _seg_sums(lab, fiedler, m):
    num = _BIG + 1
    flat = lab.reshape(-1)
    ys, xs = jnp.meshgrid(jnp.arange(_H, dtype=jnp.float32),
                          jnp.arange(_W, dtype=jnp.float32), indexing='ij')
    yf = ys.reshape(-1)
    xf = xs.reshape(-1)
    f = fiedler.reshape(-1)
    mf_ = m.reshape(-1)
    data = jnp.stack([mf_, yf * mf_, xf * mf_, f * mf_, f * f * mf_,
                      (yf * yf + xf * xf) * mf_], axis=-1)
    return jax.ops.segment_sum(data, flat, num)


def _stats_from_sums(s):
    cnt = s[:, 0]
    c = jnp.maximum(cnt, 1.0)
    cy = s[:, 1] / c
    cx = s[:, 2] / c
    mf = s[:, 3] / c
    vf = jnp.maximum(s[:, 4] / c - mf * mf, 0.0)
    g2 = jnp.maximum(s[:, 5] / c - (cy * cy + cx * cx), 0.0)
    sig = jnp.stack([mf, jnp.sqrt(vf), jnp.sqrt(cnt) / _H, jnp.sqrt(g2) / _H],
                    axis=-1)
    return cy, cx, mf, sig


def _pipeline(img_A, img_B, fiedler_A, fiedler_B, gate_A, contours_A, interpret=False):
    h, w = _H, _W
    mask_A = jnp.logical_and(contours_A, gate_A < 0.5)
    mask_B = fiedler_B > jnp.mean(fiedler_B)
    masks = jnp.stack([mask_A, mask_B]).astype(jnp.int32)
    labs = _cc_pallas(masks, interpret=interpret)
    lab_A, lab_B = labs[0], labs[1]
    mA = mask_A.astype(jnp.float32)
    mB = mask_B.astype(jnp.float32)
    sumsA = _seg_sums(lab_A, fiedler_A, mA)
    sumsB = _seg_sums(lab_B, fiedler_B, mB)
    cntA = sumsA[:, 0].at[_BIG].set(0.0)
    cntB = sumsB[:, 0].at[_BIG].set(0.0)
    selA = jnp.where(cntA >= _MIN_PIXELS, cntA, 0.0)
    selB = jnp.where(cntB >= _MIN_PIXELS, cntB, 0.0)
    valsA, idsA = jax.lax.top_k(selA, _MAX_SEGMENTS)
    valsB, idsB = jax.lax.top_k(selB, _MAX_SEGMENTS)
    validA = valsA >= float(_MIN_PIXELS)
    validB = valsB >= float(_MIN_PIXELS)
    cayT, caxT, mfAT, sA = _stats_from_sums(sumsA[idsA])
    cbyT, cbxT, _, sB = _stats_from_sums(sumsB[idsB])
    dist = jnp.sum((sA[:, None, :] - sB[None, :, :]) ** 2, axis=-1)
    dist = dist + 1e9 * (1.0 - validB.astype(jnp.float32))[None, :]
    match = jnp.argmin(dist, axis=1)
    labB_sel = idsB[match]
    okv = (validA & validB[match]).astype(jnp.float32)
    cay = cayT
    cax = caxT
    cby = cbyT[match]
    cbx = cbxT[match]
    th = mfAT * math.pi
    tsin = _TRANSLATION_STRENGTH * jnp.sin(th)
    tcos = _TRANSLATION_STRENGTH * jnp.cos(th)
    scf = jnp.stack([cay, cax, cby, cbx, tsin, tcos, okv])
    sci = labB_sel.reshape(1, _MAX_SEGMENTS)
    base4 = jnp.concatenate(
        [jnp.moveaxis(img_B, -1, 0), jnp.ones((1, h, w), jnp.float32)])
    imgA3 = jnp.moveaxis(img_A, -1, 0)
    out3 = _raster_pallas(base4, imgA3, lab_B, scf, sci, interpret=interpret)
    return jnp.moveaxis(out3, 0, -1)


def kernel(img_A, img_B, fiedler_A, fiedler_B, gate_A, contours_A):
    return _pipeline(img_A, img_B, fiedler_A, fiedler_B, gate_A, contours_A)
